# Initial kernel scaffold; baseline (speedup 1.0000x reference)
import functools

import jax
import jax.numpy as jnp
from jax import lax
from jax.experimental import pallas as pl
from jax.experimental.pallas import tpu as pltpu
from jax.experimental.pallas import tpu_sc as plsc

_NC = 2
_NS = 16
_NW = _NC * _NS
_CHUNK = 128
_ZROWS = 64
_BN = 1024


def _round_up(a, b):
    return (a + b - 1) // b * b


def _sc_pass(n_pad, w, rpw, u, src2d, dst2d):
    gather = u is not None
    mesh = plsc.VectorSubcoreMesh(core_axis_name="c", subcore_axis_name="s")
    stripe = n_pad // _NS
    n_zdma = stripe // _ZROWS
    out_type = jax.ShapeDtypeStruct((_NC, n_pad, w), jnp.float32)

    scratch = [
        pltpu.VMEM((rpw, _CHUNK), jnp.int32),
        pltpu.VMEM((_CHUNK, w), jnp.float32),
        pltpu.VMEM((_ZROWS, w), jnp.float32),
        pltpu.VMEM_SHARED((n_pad, w), jnp.float32),
        pltpu.SemaphoreType.DMA,
    ]
    if gather:
        scratch.insert(0, pltpu.VMEM((rpw, _CHUNK), jnp.int32))

    def body(u_hbm, src_hbm, dst_hbm, out_hbm, src_v, dst_v, rows_v, zbuf,
             acc, sem):
        cid = lax.axis_index("c")
        sid = lax.axis_index("s")
        gw = cid * _NS + sid

        zvec = jnp.zeros((16,), jnp.float32)

        @pl.loop(0, _ZROWS)
        def _(i):
            for c in range(w // 16):
                zbuf[i, pl.ds(c * 16, 16)] = zvec

        if not gather:
            ones = jnp.ones((16,), jnp.float32)

            @pl.loop(0, _CHUNK)
            def _(i):
                for c in range(w // 16):
                    rows_v[i, pl.ds(c * 16, 16)] = ones

        base_r = sid * stripe

        @pl.loop(0, n_zdma)
        def _(i):
            pltpu.sync_copy(zbuf, acc.at[pl.ds(base_r + i * _ZROWS, _ZROWS)])

        ebase = gw * rpw
        pltpu.sync_copy(dst_hbm.at[pl.ds(ebase, rpw)], dst_v)
        if gather:
            pltpu.sync_copy(src_hbm.at[pl.ds(ebase, rpw)], src_v)

        plsc.subcore_barrier()

        @pl.loop(0, rpw)
        def _(j):
            if gather:
                pltpu.async_copy(u_hbm.at[src_v.at[j]], rows_v, sem).wait()
            pltpu.sync_copy(rows_v, acc.at[dst_v.at[j]], add=True)

        plsc.subcore_barrier()

        pltpu.sync_copy(acc.at[pl.ds(base_r, stripe)],
                        out_hbm.at[cid].at[pl.ds(base_r, stripe)])

    if gather:
        @functools.partial(pl.kernel, out_type=out_type, mesh=mesh,
                           scratch_types=scratch)
        def k(u_hbm, src_hbm, dst_hbm, out_hbm, src_v, dst_v, rows_v, zbuf,
              acc, sem):
            body(u_hbm, src_hbm, dst_hbm, out_hbm, src_v, dst_v, rows_v,
                 zbuf, acc, sem)

        return k(u, src2d, dst2d)
    else:
        @functools.partial(pl.kernel, out_type=out_type, mesh=mesh,
                           scratch_types=scratch)
        def k(dst_hbm, out_hbm, dst_v, rows_v, zbuf, acc, sem):
            body(None, None, dst_hbm, out_hbm, None, dst_v, rows_v, zbuf,
                 acc, sem)

        return k(dst2d)


def _tc_mm(x, W):
    n_pad, d = x.shape
    w = W.shape[1]

    def body(x_ref, w_ref, o_ref):
        o_ref[...] = jnp.dot(x_ref[...], w_ref[...],
                             preferred_element_type=jnp.float32)

    return pl.pallas_call(
        body,
        grid=(n_pad // _BN,),
        in_specs=[
            pl.BlockSpec((_BN, d), lambda i: (i, 0)),
            pl.BlockSpec((d, w), lambda i: (0, 0)),
        ],
        out_specs=pl.BlockSpec((_BN, w), lambda i: (i, 0)),
        out_shape=jax.ShapeDtypeStruct((n_pad, w), jnp.float32),
    )(x, W)


def _tc_dinv_u1(pdeg, h1):
    n_pad, w = h1.shape
    wd = pdeg.shape[2]

    def body(p_ref, h_ref, dinv_ref, u1_ref):
        deg = p_ref[0, :, 0:1] + p_ref[1, :, 0:1] + 1.0
        dinv = lax.rsqrt(jnp.maximum(deg, 1e-12))
        dinv_ref[...] = dinv
        u1_ref[...] = h_ref[...] * dinv

    return pl.pallas_call(
        body,
        grid=(n_pad // _BN,),
        in_specs=[
            pl.BlockSpec((2, _BN, wd), lambda i: (0, i, 0)),
            pl.BlockSpec((_BN, w), lambda i: (i, 0)),
        ],
        out_specs=[
            pl.BlockSpec((_BN, 1), lambda i: (i, 0)),
            pl.BlockSpec((_BN, w), lambda i: (i, 0)),
        ],
        out_shape=[
            jax.ShapeDtypeStruct((n_pad, 1), jnp.float32),
            jax.ShapeDtypeStruct((n_pad, w), jnp.float32),
        ],
    )(pdeg, h1)


def _tc_combine(p, u, dinv, b, Wn, relu):
    n_pad, w = u.shape
    wn = Wn.shape[1]
    b2 = b.reshape(1, w)

    def body(p_ref, u_ref, dinv_ref, b_ref, w_ref, o_ref):
        s = (p_ref[0] + p_ref[1] + u_ref[...]) * dinv_ref[...] + b_ref[...]
        if relu:
            s = jnp.maximum(s, 0.0)
        o_ref[...] = jnp.dot(s, w_ref[...],
                             preferred_element_type=jnp.float32) * dinv_ref[...]

    return pl.pallas_call(
        body,
        grid=(n_pad // _BN,),
        in_specs=[
            pl.BlockSpec((2, _BN, w), lambda i: (0, i, 0)),
            pl.BlockSpec((_BN, w), lambda i: (i, 0)),
            pl.BlockSpec((_BN, 1), lambda i: (i, 0)),
            pl.BlockSpec((1, w), lambda i: (0, 0)),
            pl.BlockSpec((w, wn), lambda i: (0, 0)),
        ],
        out_specs=pl.BlockSpec((_BN, wn), lambda i: (i, 0)),
        out_shape=jax.ShapeDtypeStruct((n_pad, wn), jnp.float32),
    )(p, u, dinv, b2, Wn)


def _tc_final(p, u, dinv, b4):
    n_pad, w = u.shape
    b2 = b4.reshape(1, 1)

    def body(p_ref, u_ref, dinv_ref, b_ref, o_ref):
        s = (p_ref[0, :, 0:1] + p_ref[1, :, 0:1] + u_ref[:, 0:1]) \
            * dinv_ref[...] + b_ref[...]
        o_ref[...] = jax.nn.sigmoid(s)

    return pl.pallas_call(
        body,
        grid=(n_pad // _BN,),
        in_specs=[
            pl.BlockSpec((2, _BN, w), lambda i: (0, i, 0)),
            pl.BlockSpec((_BN, w), lambda i: (i, 0)),
            pl.BlockSpec((_BN, 1), lambda i: (i, 0)),
            pl.BlockSpec((1, 1), lambda i: (0, 0)),
        ],
        out_specs=pl.BlockSpec((_BN, 1), lambda i: (i, 0)),
        out_shape=jax.ShapeDtypeStruct((n_pad, 1), jnp.float32),
    )(p, u, dinv, b2)


def kernel(x, edge_index, W1, b1, W2, b2, W3, b3, W4, b4):
    n, d_in = x.shape
    e = edge_index.shape[1]

    n_pad = _round_up(n, _NS * _ZROWS)
    n_pad = _round_up(n_pad, _BN)
    e_pad = _round_up(e, _NW * _CHUNK)
    rpw = e_pad // (_NW * _CHUNK)

    src = edge_index[0]
    dst = edge_index[1]
    pad_e = e_pad - e
    src2d = jnp.concatenate(
        [src, jnp.zeros((pad_e,), jnp.int32)]).reshape(-1, _CHUNK)
    dst2d = jnp.concatenate(
        [dst, jnp.full((pad_e,), n, jnp.int32)]).reshape(-1, _CHUNK)

    x_pad = jnp.pad(x, ((0, n_pad - n), (0, 0)))
    W4p = jnp.pad(W4, ((0, 0), (0, 15)))

    pdeg = _sc_pass(n_pad, 16, rpw, None, None, dst2d)
    h1 = _tc_mm(x_pad, W1)
    dinv, u1 = _tc_dinv_u1(pdeg, h1)

    p1 = _sc_pass(n_pad, 16, rpw, u1, src2d, dst2d)
    u2 = _tc_combine(p1, u1, dinv, b1, W2, relu=True)

    p2 = _sc_pass(n_pad, 32, rpw, u2, src2d, dst2d)
    u3 = _tc_combine(p2, u2, dinv, b2, W3, relu=True)

    p3 = _sc_pass(n_pad, 32, rpw, u3, src2d, dst2d)
    u4 = _tc_combine(p3, u3, dinv, b3, W4p, relu=False)

    p4 = _sc_pass(n_pad, 16, rpw, u4, src2d, dst2d)
    out = _tc_final(p4, u4, dinv, b4)

    return out[:n]


# speedup vs baseline: 16.4925x; 16.4925x over previous
import functools

import jax
import jax.numpy as jnp
from jax import lax
from jax.experimental import pallas as pl
from jax.experimental.pallas import tpu as pltpu
from jax.experimental.pallas import tpu_sc as plsc

_NC = 2
_NS = 16
_NW = _NC * _NS
_CHUNK = 128
_ZROWS = 64
_BN = 1024


def _round_up(a, b):
    return (a + b - 1) // b * b


def _sc_pass(n_pad, w, rpw, u, src2d, dst2d):
    gather = u is not None
    mesh = plsc.VectorSubcoreMesh(core_axis_name="c", subcore_axis_name="s")
    cparams = pltpu.CompilerParams(use_tc_tiling_on_sc=False)
    stripe = n_pad // _NS
    n_zdma = stripe // _ZROWS
    out_type = jax.ShapeDtypeStruct((_NC, n_pad, w), jnp.float32)

    kb = 8
    scratch = [
        pltpu.VMEM((kb, _CHUNK), jnp.int32),
        pltpu.VMEM((_CHUNK, w), jnp.float32),
        pltpu.VMEM((_ZROWS, w), jnp.float32),
        pltpu.VMEM_SHARED((n_pad, w), jnp.float32),
        pltpu.SemaphoreType.DMA,
    ]
    if gather:
        scratch.insert(0, pltpu.VMEM((kb, _CHUNK), jnp.int32))

    def body(u_hbm, src_hbm, dst_hbm, out_hbm, src_v, dst_v, rows_v, zbuf,
             acc, sem):
        cid = lax.axis_index("c")
        sid = lax.axis_index("s")
        gw = cid * _NS + sid

        zvec = jnp.zeros((16,), jnp.float32)

        @pl.loop(0, _ZROWS)
        def _(i):
            for c in range(w // 16):
                zbuf[i, pl.ds(c * 16, 16)] = zvec

        if not gather:
            ones = jnp.ones((16,), jnp.float32)

            @pl.loop(0, _CHUNK)
            def _(i):
                for c in range(w // 16):
                    rows_v[i, pl.ds(c * 16, 16)] = ones

        base_r = sid * stripe

        @pl.loop(0, n_zdma)
        def _(i):
            pltpu.sync_copy(zbuf, acc.at[pl.ds(base_r + i * _ZROWS, _ZROWS)])

        plsc.subcore_barrier()

        ebase = gw * rpw

        @pl.loop(0, rpw // kb)
        def _(blk):
            pltpu.sync_copy(dst_hbm.at[pl.ds(ebase + blk * kb, kb)], dst_v)
            if gather:
                pltpu.sync_copy(src_hbm.at[pl.ds(ebase + blk * kb, kb)],
                                src_v)

            @pl.loop(0, kb)
            def _(j):
                if gather:
                    pltpu.async_copy(u_hbm.at[src_v.at[j]], rows_v,
                                     sem).wait()
                pltpu.sync_copy(rows_v, acc.at[dst_v.at[j]], add=True)

        plsc.subcore_barrier()

        pltpu.sync_copy(acc.at[pl.ds(base_r, stripe)],
                        out_hbm.at[cid].at[pl.ds(base_r, stripe)])

    if gather:
        @functools.partial(pl.kernel, out_type=out_type, mesh=mesh,
                           scratch_types=scratch, compiler_params=cparams)
        def k(u_hbm, src_hbm, dst_hbm, out_hbm, src_v, dst_v, rows_v, zbuf,
              acc, sem):
            body(u_hbm, src_hbm, dst_hbm, out_hbm, src_v, dst_v, rows_v,
                 zbuf, acc, sem)

        return k(u, src2d, dst2d)
    else:
        @functools.partial(pl.kernel, out_type=out_type, mesh=mesh,
                           scratch_types=scratch, compiler_params=cparams)
        def k(dst_hbm, out_hbm, dst_v, rows_v, zbuf, acc, sem):
            body(None, None, dst_hbm, out_hbm, None, dst_v, rows_v, zbuf,
                 acc, sem)

        return k(dst2d)


def _tc_mm(x, W):
    n_pad, d = x.shape
    w = W.shape[1]

    def body(x_ref, w_ref, o_ref):
        o_ref[...] = jnp.dot(x_ref[...], w_ref[...],
                             preferred_element_type=jnp.float32)

    return pl.pallas_call(
        body,
        grid=(n_pad // _BN,),
        in_specs=[
            pl.BlockSpec((_BN, d), lambda i: (i, 0)),
            pl.BlockSpec((d, w), lambda i: (0, 0)),
        ],
        out_specs=pl.BlockSpec((_BN, w), lambda i: (i, 0)),
        out_shape=jax.ShapeDtypeStruct((n_pad, w), jnp.float32),
    )(x, W)


def _tc_dinv_u1(pdeg, h1):
    n_pad, w = h1.shape
    wd = pdeg.shape[2]

    def body(p_ref, h_ref, dinv_ref, u1_ref):
        deg = p_ref[0, :, 0:1] + p_ref[1, :, 0:1] + 1.0
        dinv = lax.rsqrt(jnp.maximum(deg, 1e-12))
        dinv_ref[...] = dinv
        u1_ref[...] = h_ref[...] * dinv

    return pl.pallas_call(
        body,
        grid=(n_pad // _BN,),
        in_specs=[
            pl.BlockSpec((2, _BN, wd), lambda i: (0, i, 0)),
            pl.BlockSpec((_BN, w), lambda i: (i, 0)),
        ],
        out_specs=[
            pl.BlockSpec((_BN, 1), lambda i: (i, 0)),
            pl.BlockSpec((_BN, w), lambda i: (i, 0)),
        ],
        out_shape=[
            jax.ShapeDtypeStruct((n_pad, 1), jnp.float32),
            jax.ShapeDtypeStruct((n_pad, w), jnp.float32),
        ],
    )(pdeg, h1)


def _tc_combine(p, u, dinv, b, Wn, relu):
    n_pad, w = u.shape
    wn = Wn.shape[1]
    b2 = b.reshape(1, w)

    def body(p_ref, u_ref, dinv_ref, b_ref, w_ref, o_ref):
        s = (p_ref[0] + p_ref[1] + u_ref[...]) * dinv_ref[...] + b_ref[...]
        if relu:
            s = jnp.maximum(s, 0.0)
        o_ref[...] = jnp.dot(s, w_ref[...],
                             preferred_element_type=jnp.float32) * dinv_ref[...]

    return pl.pallas_call(
        body,
        grid=(n_pad // _BN,),
        in_specs=[
            pl.BlockSpec((2, _BN, w), lambda i: (0, i, 0)),
            pl.BlockSpec((_BN, w), lambda i: (i, 0)),
            pl.BlockSpec((_BN, 1), lambda i: (i, 0)),
            pl.BlockSpec((1, w), lambda i: (0, 0)),
            pl.BlockSpec((w, wn), lambda i: (0, 0)),
        ],
        out_specs=pl.BlockSpec((_BN, wn), lambda i: (i, 0)),
        out_shape=jax.ShapeDtypeStruct((n_pad, wn), jnp.float32),
    )(p, u, dinv, b2, Wn)


def _tc_final(p, u, dinv, b4):
    n_pad, w = u.shape
    b2 = b4.reshape(1, 1)

    def body(p_ref, u_ref, dinv_ref, b_ref, o_ref):
        s = (p_ref[0, :, 0:1] + p_ref[1, :, 0:1] + u_ref[:, 0:1]) \
            * dinv_ref[...] + b_ref[...]
        o_ref[...] = jax.nn.sigmoid(s)

    return pl.pallas_call(
        body,
        grid=(n_pad // _BN,),
        in_specs=[
            pl.BlockSpec((2, _BN, w), lambda i: (0, i, 0)),
            pl.BlockSpec((_BN, w), lambda i: (i, 0)),
            pl.BlockSpec((_BN, 1), lambda i: (i, 0)),
            pl.BlockSpec((1, 1), lambda i: (0, 0)),
        ],
        out_specs=pl.BlockSpec((_BN, 1), lambda i: (i, 0)),
        out_shape=jax.ShapeDtypeStruct((n_pad, 1), jnp.float32),
    )(p, u, dinv, b2)


def kernel(x, edge_index, W1, b1, W2, b2, W3, b3, W4, b4):
    n, d_in = x.shape
    e = edge_index.shape[1]

    n_pad = _round_up(n, _NS * _ZROWS)
    n_pad = _round_up(n_pad, _BN)
    e_pad = _round_up(e, _NW * _CHUNK * 8)
    rpw = e_pad // (_NW * _CHUNK)

    src = edge_index[0]
    dst = edge_index[1]
    pad_e = e_pad - e
    src2d = jnp.concatenate(
        [src, jnp.zeros((pad_e,), jnp.int32)]).reshape(-1, _CHUNK)
    dst2d = jnp.concatenate(
        [dst, jnp.full((pad_e,), n, jnp.int32)]).reshape(-1, _CHUNK)

    x_pad = jnp.pad(x, ((0, n_pad - n), (0, 0)))
    W4p = jnp.pad(W4, ((0, 0), (0, 15)))

    pdeg = _sc_pass(n_pad, 16, rpw, None, None, dst2d)
    h1 = _tc_mm(x_pad, W1)
    dinv, u1 = _tc_dinv_u1(pdeg, h1)

    p1 = _sc_pass(n_pad, 16, rpw, u1, src2d, dst2d)
    u2 = _tc_combine(p1, u1, dinv, b1, W2, relu=True)

    p2 = _sc_pass(n_pad, 32, rpw, u2, src2d, dst2d)
    u3 = _tc_combine(p2, u2, dinv, b2, W3, relu=True)

    p3 = _sc_pass(n_pad, 32, rpw, u3, src2d, dst2d)
    u4 = _tc_combine(p3, u3, dinv, b3, W4p, relu=False)

    p4 = _sc_pass(n_pad, 16, rpw, u4, src2d, dst2d)
    out = _tc_final(p4, u4, dinv, b4)

    return out[:n]
